# Initial kernel scaffold; baseline (speedup 1.0000x reference)
import jax
import jax.numpy as jnp
from jax.experimental import pallas as pl


def kernel(x, edge_index, W, b, gamma, beta):
    raise NotImplementedError("write your pallas kernel here")


# speedup vs baseline: 18.1636x; 18.1636x over previous
import functools

import jax
import jax.numpy as jnp
from jax import lax
from jax.experimental import pallas as pl
from jax.experimental.pallas import tpu as pltpu
from jax.experimental.pallas import tpu_sc as plsc

N = 10000
E = 160000
D = 256
H = 128
NC = 2
NS = 16
NPAD = 10240
CHW = NPAD // NS
EPT_A = E // (NC * NS)
EPT_C = E // NS
CH = 80
NCH = EPT_C // CH
WR = 80
WR_PT = 8
ZR = 40
ZB_PT = 16


def _sc_degree(dst_i32):
    mesh = plsc.VectorSubcoreMesh(core_axis_name="c", subcore_axis_name="s")

    @functools.partial(
        pl.kernel,
        out_type=jax.ShapeDtypeStruct((NC, NPAD), jnp.float32),
        mesh=mesh,
        compiler_params=pltpu.CompilerParams(needs_layout_passes=False),
        scratch_types=[
            pltpu.VMEM((EPT_A + 16,), jnp.int32),
            pltpu.VMEM((NPAD,), jnp.float32),
            pltpu.VMEM_SHARED((NS, NPAD), jnp.float32),
            pltpu.VMEM((CHW,), jnp.float32),
            pltpu.VMEM((CHW,), jnp.float32),
        ],
    )
    def k(dst_hbm, out_hbm, idxv, hist, shared, accv, tmpv):
        c = lax.axis_index("c")
        s = lax.axis_index("s")
        tid = c * NS + s
        zero16 = jnp.zeros((16,), jnp.float32)
        zero16i = jnp.zeros((16,), jnp.int32)
        idxv[pl.ds(EPT_A - 8, 16)] = zero16i
        pltpu.sync_copy(dst_hbm.at[pl.ds(tid * EPT_A, EPT_A)],
                        idxv.at[pl.ds(0, EPT_A)])

        def zloop(i, _):
            hist[pl.ds(i * 16, 16)] = zero16
            return ()
        lax.fori_loop(0, NPAD // 16, zloop, ())

        ones = jnp.ones((16,), jnp.float32)
        lane = lax.broadcasted_iota(jnp.int32, (16,), 0)

        def hloop(i, _):
            idx = idxv[pl.ds(i * 16, 16)]
            vals = jnp.where(i * 16 + lane < EPT_A, ones, zero16)
            plsc.addupdate_scatter(hist, [idx], vals)
            return ()
        lax.fori_loop(0, (EPT_A + 15) // 16, hloop, ())

        pltpu.sync_copy(hist, shared.at[s])
        plsc.subcore_barrier()

        def z2(i, _):
            accv[pl.ds(i * 16, 16)] = zero16
            return ()
        lax.fori_loop(0, CHW // 16, z2, ())
        for t in range(NS):
            pltpu.sync_copy(shared.at[t, pl.ds(s * CHW, CHW)], tmpv)

            def aloop(i, _):
                accv[pl.ds(i * 16, 16)] = (accv[pl.ds(i * 16, 16)]
                                           + tmpv[pl.ds(i * 16, 16)])
                return ()
            lax.fori_loop(0, CHW // 16, aloop, ())
        pltpu.sync_copy(accv, out_hbm.at[c, pl.ds(s * CHW, CHW)])

    return k(dst_i32)


def _tc_linear(x, W, degp_col):
    def body(x_ref, w_ref, degp_ref, h_ref, dis_ref):
        deg = degp_ref[0] + degp_ref[1] + 1.0
        dis = lax.rsqrt(deg)
        discol = dis[:N]
        h = jnp.dot(x_ref[...], w_ref[...],
                    preferred_element_type=jnp.float32)
        hs = h * discol
        h_ref[0:N] = hs[:, 0:H]
        h_ref[N:2 * N] = hs[:, H:D]
        dis_ref[...] = discol

    return pl.pallas_call(
        body,
        out_shape=(jax.ShapeDtypeStruct((2 * N, H), jnp.float32),
                   jax.ShapeDtypeStruct((N, 1), jnp.float32)),
    )(x, W, degp_col)


def _sc_edge_pass(h2, src_i32, dst_i32):
    mesh = plsc.VectorSubcoreMesh(core_axis_name="c", subcore_axis_name="s")

    @functools.partial(
        pl.kernel,
        out_type=jax.ShapeDtypeStruct((NC, N, H), jnp.float32),
        mesh=mesh,
        compiler_params=pltpu.CompilerParams(needs_layout_passes=False),
        scratch_types=[
            pltpu.VMEM((EPT_C,), jnp.int32),
            pltpu.VMEM((EPT_C,), jnp.int32),
            pltpu.VMEM((2, CH), jnp.int32),
            pltpu.VMEM((2, CH), jnp.int32),
            pltpu.VMEM((2, CH, H), jnp.float32),
            pltpu.VMEM((ZR, H), jnp.float32),
            pltpu.VMEM_SHARED((N, H), jnp.float32),
            pltpu.SemaphoreType.DMA((2,)),
            pltpu.SemaphoreType.DMA((2,)),
        ],
    )
    def k(h_hbm, src_hbm, dst_hbm, out_hbm,
          srcv, dstv, idxg, idxs, rows, zbuf, accS, gsem, ssem):
        c = lax.axis_index("c")
        s = lax.axis_index("s")
        hoff = c * N
        base = s * EPT_C
        pltpu.sync_copy(src_hbm.at[pl.ds(base, EPT_C)], srcv)
        pltpu.sync_copy(dst_hbm.at[pl.ds(base, EPT_C)], dstv)

        zero16 = jnp.zeros((16,), jnp.float32)

        def zb(kk, _):
            r = kk // (H // 16)
            jcol = (kk % (H // 16)) * 16
            zbuf[r, pl.ds(jcol, 16)] = zero16
            return ()
        lax.fori_loop(0, ZR * (H // 16), zb, ())
        for rblk in range(ZB_PT):
            off = pl.multiple_of(s * (ZR * ZB_PT) + rblk * ZR, 8)

            @pl.when(off < N)
            def _():
                pltpu.sync_copy(zbuf, accS.at[pl.ds(off, ZR)])
        plsc.subcore_barrier()


        def build(cur, slot):
            off = cur * CH
            for j in range(CH // 16):
                idxg[slot, pl.ds(j * 16, 16)] = (
                    srcv[pl.ds(off + j * 16, 16)] + hoff)
                idxs[slot, pl.ds(j * 16, 16)] = dstv[pl.ds(off + j * 16, 16)]

        build(0, 0)
        pltpu.async_copy(h_hbm.at[idxg.at[0]], rows.at[0], gsem.at[0])

        @pl.loop(0, NCH, step=2)
        def _(g):
            for b in range(2):
                cur = g + b

                def work(cur=cur, b=b):
                    pltpu.make_async_copy(h_hbm.at[idxg.at[b]],
                                          rows.at[b], gsem.at[b]).wait()
                    pltpu.async_copy(rows.at[b], accS.at[idxs.at[b]],
                                     ssem.at[b], add=True)

                    @pl.when(cur >= 1)
                    def _():
                        pltpu.make_async_copy(
                            rows.at[1 - b], accS.at[idxs.at[1 - b]],
                            ssem.at[1 - b]).wait()

                    @pl.when(cur + 1 < NCH)
                    def _():
                        build(cur + 1, 1 - b)
                        pltpu.async_copy(h_hbm.at[idxg.at[1 - b]],
                                         rows.at[1 - b], gsem.at[1 - b])

                if b == 1:
                    pl.when(cur < NCH)(work)
                else:
                    work()

        last = (NCH - 1) % 2
        pltpu.make_async_copy(rows.at[last], accS.at[idxs.at[last]],
                              ssem.at[last]).wait()

        plsc.subcore_barrier()
        for rblk in range(WR_PT):
            off = pl.multiple_of(s * (WR * WR_PT) + rblk * WR, 8)

            @pl.when(off < N)
            def _():
                pltpu.sync_copy(accS.at[pl.ds(off, WR)],
                                out_hbm.at[c, pl.ds(off, WR)])

    return k(h2, src_i32, dst_i32)


def _tc_epilogue(acc, h2, dis, b, gamma, beta):
    def body(acc_ref, h_ref, dis_ref, b_ref, g_ref, be_ref, o_ref):
        dis_col = dis_ref[...]
        t0 = (acc_ref[0] + h_ref[0:N]) * dis_col
        t1 = (acc_ref[1] + h_ref[N:2 * N]) * dis_col
        t = jnp.concatenate([t0, t1], axis=1) + b_ref[...]
        mu = jnp.mean(t, axis=1, keepdims=True)
        d = t - mu
        var = jnp.mean(d * d, axis=1, keepdims=True)
        o_ref[...] = (d * lax.rsqrt(var + 1e-5)) * g_ref[...] + be_ref[...]

    return pl.pallas_call(
        body,
        out_shape=jax.ShapeDtypeStruct((N, D), jnp.float32),
    )(acc, h2, dis, b, gamma, beta)


def kernel(x, edge_index, W, b, gamma, beta):
    src = edge_index[0].astype(jnp.int32)
    dst = edge_index[1].astype(jnp.int32)
    degp = _sc_degree(dst)
    degp_col = degp[:, :, None]
    h2, dis = _tc_linear(x, W, degp_col)
    acc = _sc_edge_pass(h2, src, dst)
    return _tc_epilogue(acc, h2, dis,
                        b.reshape(1, D), gamma.reshape(1, D),
                        beta.reshape(1, D))


# speedup vs baseline: 23.6569x; 1.3024x over previous
import functools

import jax
import jax.numpy as jnp
from jax import lax
from jax.experimental import pallas as pl
from jax.experimental.pallas import tpu as pltpu
from jax.experimental.pallas import tpu_sc as plsc

N = 10000
E = 160000
D = 256
H = 128
NC = 2
NS = 16
NPAD = 10240
CHW = NPAD // NS
EPT_A = E // (NC * NS)
EPT_C = E // NS
CH = 80
NCH = EPT_C // CH
WR = 80
WR_PT = 8
NSLOT = 4
SEC = 2000
CPS = SEC // CH
NSEC = EPT_C // SEC


def _sc_degree(dst_i32):
    mesh = plsc.VectorSubcoreMesh(core_axis_name="c", subcore_axis_name="s")

    @functools.partial(
        pl.kernel,
        out_type=jax.ShapeDtypeStruct((NC, NPAD), jnp.float32),
        mesh=mesh,
        compiler_params=pltpu.CompilerParams(needs_layout_passes=False),
        scratch_types=[
            pltpu.VMEM((EPT_A + 16,), jnp.int32),
            pltpu.VMEM((NPAD,), jnp.float32),
            pltpu.VMEM_SHARED((NS, NPAD), jnp.float32),
            pltpu.VMEM((CHW,), jnp.float32),
            pltpu.VMEM((CHW,), jnp.float32),
        ],
    )
    def k(dst_hbm, out_hbm, idxv, hist, shared, accv, tmpv):
        c = lax.axis_index("c")
        s = lax.axis_index("s")
        tid = c * NS + s
        zero16 = jnp.zeros((16,), jnp.float32)
        zero16i = jnp.zeros((16,), jnp.int32)
        idxv[pl.ds(EPT_A - 8, 16)] = zero16i
        pltpu.sync_copy(dst_hbm.at[pl.ds(tid * EPT_A, EPT_A)],
                        idxv.at[pl.ds(0, EPT_A)])

        def zloop(i, _):
            hist[pl.ds(i * 16, 16)] = zero16
            return ()
        lax.fori_loop(0, NPAD // 16, zloop, ())

        ones = jnp.ones((16,), jnp.float32)
        lane = lax.broadcasted_iota(jnp.int32, (16,), 0)

        def hloop(i, _):
            idx = idxv[pl.ds(i * 16, 16)]
            vals = jnp.where(i * 16 + lane < EPT_A, ones, zero16)
            plsc.addupdate_scatter(hist, [idx], vals)
            return ()
        lax.fori_loop(0, (EPT_A + 15) // 16, hloop, ())

        pltpu.sync_copy(hist, shared.at[s])
        plsc.subcore_barrier()

        def z2(i, _):
            accv[pl.ds(i * 16, 16)] = zero16
            return ()
        lax.fori_loop(0, CHW // 16, z2, ())
        for t in range(NS):
            pltpu.sync_copy(shared.at[t, pl.ds(s * CHW, CHW)], tmpv)

            def aloop(i, _):
                accv[pl.ds(i * 16, 16)] = (accv[pl.ds(i * 16, 16)]
                                           + tmpv[pl.ds(i * 16, 16)])
                return ()
            lax.fori_loop(0, CHW // 16, aloop, ())
        pltpu.sync_copy(accv, out_hbm.at[c, pl.ds(s * CHW, CHW)])

    return k(dst_i32)


def _tc_linear(x, W, degp_col):
    def body(x_ref, w_ref, degp_ref, h_ref, dis_ref):
        deg = degp_ref[0] + degp_ref[1] + 1.0
        dis = lax.rsqrt(deg)
        discol = dis[:N]
        h = jnp.dot(x_ref[...], w_ref[...],
                    preferred_element_type=jnp.float32)
        hs = h * discol
        h_ref[0:N] = hs[:, 0:H]
        h_ref[N:2 * N] = hs[:, H:D]
        dis_ref[...] = discol

    return pl.pallas_call(
        body,
        out_shape=(jax.ShapeDtypeStruct((2 * N, H), jnp.float32),
                   jax.ShapeDtypeStruct((N, 1), jnp.float32)),
    )(x, W, degp_col)


def _sc_edge_pass(h2, src_i32, dst_i32):
    mesh = plsc.VectorSubcoreMesh(core_axis_name="c", subcore_axis_name="s")

    @functools.partial(
        pl.kernel,
        out_type=jax.ShapeDtypeStruct((NC, N, H), jnp.float32),
        mesh=mesh,
        compiler_params=pltpu.CompilerParams(needs_layout_passes=False),
        scratch_types=[
            pltpu.VMEM((SEC,), jnp.int32),
            pltpu.VMEM((SEC,), jnp.int32),
            pltpu.VMEM((NSLOT, CH), jnp.int32),
            pltpu.VMEM((NSLOT, CH), jnp.int32),
            pltpu.VMEM((NSLOT, CH, H), jnp.float32),
            pltpu.VMEM_SHARED((N, H), jnp.float32),
            pltpu.SemaphoreType.DMA((NSLOT,)),
            pltpu.SemaphoreType.DMA((NSLOT,)),
        ],
    )
    def k(h_hbm, src_hbm, dst_hbm, out_hbm,
          srcv, dstv, idxg, idxs, rows, accS, gsem, ssem):
        c = lax.axis_index("c")
        s = lax.axis_index("s")
        hoff = c * N
        base = s * EPT_C
        zero16 = jnp.zeros((16,), jnp.float32)

        def zb(kk, _):
            r = kk // (H // 16)
            jcol = (kk % (H // 16)) * 16
            rows[0, r, pl.ds(jcol, 16)] = zero16
            return ()
        lax.fori_loop(0, CH * (H // 16), zb, ())
        for rblk in range(WR_PT):
            off = pl.multiple_of(s * (WR * WR_PT) + rblk * WR, 8)

            @pl.when(off < N)
            def _():
                pltpu.sync_copy(rows.at[0], accS.at[pl.ds(off, WR)])

        def load_sec(sec):
            o = base + sec * SEC
            pltpu.sync_copy(src_hbm.at[pl.ds(o, SEC)], srcv)
            pltpu.sync_copy(dst_hbm.at[pl.ds(o, SEC)], dstv)

        def build(t, slot):
            loff = (t % CPS) * CH
            for j in range(CH // 16):
                idxg[slot, pl.ds(j * 16, 16)] = (
                    srcv[pl.ds(loff + j * 16, 16)] + hoff)
                idxs[slot, pl.ds(j * 16, 16)] = dstv[pl.ds(loff + j * 16, 16)]

        def issue_gather(slot):
            pltpu.async_copy(h_hbm.at[idxg.at[slot]], rows.at[slot],
                             gsem.at[slot])

        load_sec(0)
        for t0 in range(NSLOT - 1):
            build(t0, t0)
            issue_gather(t0)
        plsc.subcore_barrier()

        def step(t, slot):
            pltpu.make_async_copy(h_hbm.at[idxg.at[slot]], rows.at[slot],
                                  gsem.at[slot]).wait()
            pltpu.async_copy(rows.at[slot], accS.at[idxs.at[slot]],
                             ssem.at[slot], add=True)
            pslot = (slot - 1) % NSLOT

            @pl.when(t >= 1)
            def _():
                pltpu.make_async_copy(rows.at[pslot], accS.at[idxs.at[pslot]],
                                      ssem.at[pslot]).wait()
            nxt = t + NSLOT - 1
            nslot = (slot + NSLOT - 1) % NSLOT

            @pl.when(nxt < NCH)
            def _():
                @pl.when(nxt % CPS == 0)
                def _():
                    load_sec(nxt // CPS)
                build(nxt, nslot)
                issue_gather(nslot)

        @pl.loop(0, NCH - 1, step=NSLOT)
        def _(g):
            for j in range(NSLOT):
                step(g + j, j)

        lastslot = (NCH - 1) % NSLOT
        step(NCH - 1, lastslot)
        pltpu.make_async_copy(rows.at[lastslot], accS.at[idxs.at[lastslot]],
                              ssem.at[lastslot]).wait()

        plsc.subcore_barrier()
        for rblk in range(WR_PT):
            off = pl.multiple_of(s * (WR * WR_PT) + rblk * WR, 8)

            @pl.when(off < N)
            def _():
                pltpu.sync_copy(accS.at[pl.ds(off, WR)],
                                out_hbm.at[c, pl.ds(off, WR)])

    return k(h2, src_i32, dst_i32)


def _tc_epilogue(acc, h2, dis, b, gamma, beta):
    def body(acc_ref, h_ref, dis_ref, b_ref, g_ref, be_ref, o_ref):
        dis_col = dis_ref[...]
        t0 = (acc_ref[0] + h_ref[0:N]) * dis_col
        t1 = (acc_ref[1] + h_ref[N:2 * N]) * dis_col
        t = jnp.concatenate([t0, t1], axis=1) + b_ref[...]
        mu = jnp.mean(t, axis=1, keepdims=True)
        d = t - mu
        var = jnp.mean(d * d, axis=1, keepdims=True)
        o_ref[...] = (d * lax.rsqrt(var + 1e-5)) * g_ref[...] + be_ref[...]

    return pl.pallas_call(
        body,
        out_shape=jax.ShapeDtypeStruct((N, D), jnp.float32),
    )(acc, h2, dis, b, gamma, beta)


def kernel(x, edge_index, W, b, gamma, beta):
    src = edge_index[0].astype(jnp.int32)
    dst = edge_index[1].astype(jnp.int32)
    degp = _sc_degree(dst)
    degp_col = degp[:, :, None]
    h2, dis = _tc_linear(x, W, degp_col)
    acc = _sc_edge_pass(h2, src, dst)
    return _tc_epilogue(acc, h2, dis,
                        b.reshape(1, D), gamma.reshape(1, D),
                        beta.reshape(1, D))


# speedup vs baseline: 26.4325x; 1.1173x over previous
import functools

import jax
import jax.numpy as jnp
from jax import lax
from jax.experimental import pallas as pl
from jax.experimental.pallas import tpu as pltpu
from jax.experimental.pallas import tpu_sc as plsc

N = 10000
E = 160000
D = 256
H = 128
NC = 2
NS = 16
NPAD = 10240
CHW = NPAD // NS
EPT_A = E // (NC * NS)
EPT_C = E // NS
CH = 80
NCH = EPT_C // CH
WR = 80
WR_PT = 8
NSLOT = 4
SEC = 2000
CPS = SEC // CH
NSEC = EPT_C // SEC


def _sc_degree(edge_index):
    mesh = plsc.VectorSubcoreMesh(core_axis_name="c", subcore_axis_name="s")

    @functools.partial(
        pl.kernel,
        out_type=jax.ShapeDtypeStruct((NC, NPAD), jnp.float32),
        mesh=mesh,
        compiler_params=pltpu.CompilerParams(needs_layout_passes=False),
        scratch_types=[
            pltpu.VMEM((EPT_A + 16,), jnp.int32),
            pltpu.VMEM((NPAD,), jnp.float32),
            pltpu.VMEM_SHARED((NS, NPAD), jnp.float32),
            pltpu.VMEM((CHW,), jnp.float32),
            pltpu.VMEM((CHW,), jnp.float32),
        ],
    )
    def k(ei_hbm, out_hbm, idxv, hist, shared, accv, tmpv):
        c = lax.axis_index("c")
        s = lax.axis_index("s")
        tid = c * NS + s
        zero16 = jnp.zeros((16,), jnp.float32)
        zero16i = jnp.zeros((16,), jnp.int32)
        idxv[pl.ds(EPT_A - 8, 16)] = zero16i
        pltpu.sync_copy(ei_hbm.at[pl.ds(E + tid * EPT_A, EPT_A)],
                        idxv.at[pl.ds(0, EPT_A)])

        def zloop(i, _):
            hist[pl.ds(i * 16, 16)] = zero16
            return ()
        lax.fori_loop(0, NPAD // 16, zloop, ())

        ones = jnp.ones((16,), jnp.float32)
        lane = lax.broadcasted_iota(jnp.int32, (16,), 0)

        def hloop(i, _):
            idx = idxv[pl.ds(i * 16, 16)]
            vals = jnp.where(i * 16 + lane < EPT_A, ones, zero16)
            plsc.addupdate_scatter(hist, [idx], vals)
            return ()
        lax.fori_loop(0, (EPT_A + 15) // 16, hloop, ())

        pltpu.sync_copy(hist, shared.at[s])
        plsc.subcore_barrier()

        def z2(i, _):
            accv[pl.ds(i * 16, 16)] = zero16
            return ()
        lax.fori_loop(0, CHW // 16, z2, ())
        for t in range(NS):
            pltpu.sync_copy(shared.at[t, pl.ds(s * CHW, CHW)], tmpv)

            def aloop(i, _):
                accv[pl.ds(i * 16, 16)] = (accv[pl.ds(i * 16, 16)]
                                           + tmpv[pl.ds(i * 16, 16)])
                return ()
            lax.fori_loop(0, CHW // 16, aloop, ())
        pltpu.sync_copy(accv, out_hbm.at[c, pl.ds(s * CHW, CHW)])

    return k(edge_index)


def _tc_linear(x, W, degp):
    def body(x_ref, w_ref, degp_ref, h_ref, dis_ref):
        deg = degp_ref[0] + degp_ref[1] + 1.0
        dis = lax.rsqrt(deg)
        dis_ref[...] = dis
        discol = jnp.reshape(dis, (NPAD, 1))[:N]
        h = jnp.dot(x_ref[...].astype(jnp.bfloat16),
                    w_ref[...].astype(jnp.bfloat16),
                    preferred_element_type=jnp.float32)
        hs = h * discol
        h_ref[0:N] = hs[:, 0:H]
        h_ref[N:2 * N] = hs[:, H:D]

    return pl.pallas_call(
        body,
        out_shape=(jax.ShapeDtypeStruct((2 * N, H), jnp.float32),
                   jax.ShapeDtypeStruct((NPAD,), jnp.float32)),
    )(x, W, degp)


def _sc_edge_pass(h2, edge_index):
    mesh = plsc.VectorSubcoreMesh(core_axis_name="c", subcore_axis_name="s")

    @functools.partial(
        pl.kernel,
        out_type=jax.ShapeDtypeStruct((NC, N, H), jnp.float32),
        mesh=mesh,
        compiler_params=pltpu.CompilerParams(needs_layout_passes=False),
        scratch_types=[
            pltpu.VMEM((SEC,), jnp.int32),
            pltpu.VMEM((SEC,), jnp.int32),
            pltpu.VMEM((NSLOT, CH), jnp.int32),
            pltpu.VMEM((NSLOT, CH), jnp.int32),
            pltpu.VMEM((NSLOT, CH, H), jnp.float32),
            pltpu.VMEM_SHARED((N, H), jnp.float32),
            pltpu.SemaphoreType.DMA((NSLOT,)),
            pltpu.SemaphoreType.DMA((NSLOT,)),
        ],
    )
    def k(h_hbm, ei_hbm, out_hbm,
          srcv, dstv, idxg, idxs, rows, accS, gsem, ssem):
        c = lax.axis_index("c")
        s = lax.axis_index("s")
        hoff = c * N
        base = s * EPT_C
        zero16 = jnp.zeros((16,), jnp.float32)

        def zb(kk, _):
            r = kk // (H // 16)
            jcol = (kk % (H // 16)) * 16
            rows[0, r, pl.ds(jcol, 16)] = zero16
            return ()
        lax.fori_loop(0, CH * (H // 16), zb, ())
        for rblk in range(WR_PT):
            off = pl.multiple_of(s * (WR * WR_PT) + rblk * WR, 8)

            @pl.when(off < N)
            def _():
                pltpu.sync_copy(rows.at[0], accS.at[pl.ds(off, WR)])

        def load_sec(sec):
            o = base + sec * SEC
            pltpu.sync_copy(ei_hbm.at[pl.ds(o, SEC)], srcv)
            pltpu.sync_copy(ei_hbm.at[pl.ds(E + o, SEC)], dstv)

        def build(t, slot):
            loff = (t % CPS) * CH
            for j in range(CH // 16):
                idxg[slot, pl.ds(j * 16, 16)] = (
                    srcv[pl.ds(loff + j * 16, 16)] + hoff)
                idxs[slot, pl.ds(j * 16, 16)] = dstv[pl.ds(loff + j * 16, 16)]

        def issue_gather(slot):
            pltpu.async_copy(h_hbm.at[idxg.at[slot]], rows.at[slot],
                             gsem.at[slot])

        load_sec(0)
        for t0 in range(NSLOT - 1):
            build(t0, t0)
            issue_gather(t0)
        plsc.subcore_barrier()

        def step(t, slot):
            pltpu.make_async_copy(h_hbm.at[idxg.at[slot]], rows.at[slot],
                                  gsem.at[slot]).wait()
            pltpu.async_copy(rows.at[slot], accS.at[idxs.at[slot]],
                             ssem.at[slot], add=True)
            pslot = (slot - 1) % NSLOT

            @pl.when(t >= 1)
            def _():
                pltpu.make_async_copy(rows.at[pslot], accS.at[idxs.at[pslot]],
                                      ssem.at[pslot]).wait()
            nxt = t + NSLOT - 1
            nslot = (slot + NSLOT - 1) % NSLOT

            @pl.when(nxt < NCH)
            def _():
                @pl.when(nxt % CPS == 0)
                def _():
                    load_sec(nxt // CPS)
                build(nxt, nslot)
                issue_gather(nslot)

        @pl.loop(0, NCH - 1, step=NSLOT)
        def _(g):
            for j in range(NSLOT):
                step(g + j, j)

        lastslot = (NCH - 1) % NSLOT
        step(NCH - 1, lastslot)
        pltpu.make_async_copy(rows.at[lastslot], accS.at[idxs.at[lastslot]],
                              ssem.at[lastslot]).wait()

        plsc.subcore_barrier()
        for rblk in range(WR_PT):
            off = pl.multiple_of(s * (WR * WR_PT) + rblk * WR, 8)

            @pl.when(off < N)
            def _():
                pltpu.sync_copy(accS.at[pl.ds(off, WR)],
                                out_hbm.at[c, pl.ds(off, WR)])

    return k(h2, edge_index)


def _tc_epilogue(acc, h2, dis, b, gamma, beta):
    def body(acc_ref, h_ref, dis_ref, b_ref, g_ref, be_ref, o_ref):
        dis_col = jnp.reshape(dis_ref[...], (NPAD, 1))[:N]
        t0 = (acc_ref[0] + h_ref[0:N]) * dis_col
        t1 = (acc_ref[1] + h_ref[N:2 * N]) * dis_col
        t = jnp.concatenate([t0, t1], axis=1) + b_ref[...]
        mu = jnp.mean(t, axis=1, keepdims=True)
        d = t - mu
        var = jnp.mean(d * d, axis=1, keepdims=True)
        o_ref[...] = (d * lax.rsqrt(var + 1e-5)) * g_ref[...] + be_ref[...]

    return pl.pallas_call(
        body,
        out_shape=jax.ShapeDtypeStruct((N, D), jnp.float32),
    )(acc, h2, dis, b, gamma, beta)


def kernel(x, edge_index, W, b, gamma, beta):
    ei = edge_index.astype(jnp.int32).reshape(-1)
    degp = _sc_degree(ei)
    h2, dis = _tc_linear(x, W, degp)
    acc = _sc_edge_pass(h2, ei)
    return _tc_epilogue(acc, h2, dis,
                        b.reshape(1, D), gamma.reshape(1, D),
                        beta.reshape(1, D))


# speedup vs baseline: 26.8211x; 1.0147x over previous
import functools

import jax
import jax.numpy as jnp
from jax import lax
from jax.experimental import pallas as pl
from jax.experimental.pallas import tpu as pltpu
from jax.experimental.pallas import tpu_sc as plsc

N = 10000
E = 160000
D = 256
H = 128
NC = 2
NS = 16
NPAD = 10240
CHW = NPAD // NS
EPT_A = E // (NC * NS)
EPT_C = E // NS
CH = 80
NCH = EPT_C // CH
WR = 80
WR_PT = 8
NSLOT = 4
SEC = 2000
CPS = SEC // CH
NSEC = EPT_C // SEC


def _sc_degree(edge_index):
    mesh = plsc.VectorSubcoreMesh(core_axis_name="c", subcore_axis_name="s")

    @functools.partial(
        pl.kernel,
        out_type=jax.ShapeDtypeStruct((NC, NPAD), jnp.float32),
        mesh=mesh,
        compiler_params=pltpu.CompilerParams(needs_layout_passes=False),
        scratch_types=[
            pltpu.VMEM((EPT_A + 16,), jnp.int32),
            pltpu.VMEM((NPAD,), jnp.float32),
            pltpu.VMEM_SHARED((NS, NPAD), jnp.float32),
            pltpu.VMEM((CHW,), jnp.float32),
            pltpu.VMEM((NS, CHW), jnp.float32),
            pltpu.SemaphoreType.DMA,
        ],
    )
    def k(ei_hbm, out_hbm, idxv, hist, shared, accv, tmpv, csem):
        c = lax.axis_index("c")
        s = lax.axis_index("s")
        tid = c * NS + s
        zero16 = jnp.zeros((16,), jnp.float32)
        zero16i = jnp.zeros((16,), jnp.int32)
        idxv[pl.ds(EPT_A - 8, 16)] = zero16i
        pltpu.sync_copy(ei_hbm.at[pl.ds(E + tid * EPT_A, EPT_A)],
                        idxv.at[pl.ds(0, EPT_A)])

        def zloop(i, _):
            hist[pl.ds(i * 16, 16)] = zero16
            return ()
        lax.fori_loop(0, NPAD // 16, zloop, ())

        ones = jnp.ones((16,), jnp.float32)
        lane = lax.broadcasted_iota(jnp.int32, (16,), 0)

        def hloop(i, _):
            idx = idxv[pl.ds(i * 16, 16)]
            vals = jnp.where(i * 16 + lane < EPT_A, ones, zero16)
            plsc.addupdate_scatter(hist, [idx], vals)
            return ()
        lax.fori_loop(0, (EPT_A + 15) // 16, hloop, ())

        pltpu.sync_copy(hist, shared.at[s])
        plsc.subcore_barrier()

        def z2(i, _):
            accv[pl.ds(i * 16, 16)] = zero16
            return ()
        lax.fori_loop(0, CHW // 16, z2, ())
        for t in range(NS):
            pltpu.async_copy(shared.at[t, pl.ds(s * CHW, CHW)], tmpv.at[t],
                             csem)
        for t in range(NS):
            pltpu.make_async_copy(shared.at[t, pl.ds(s * CHW, CHW)],
                                  tmpv.at[t], csem).wait()

            def aloop(i, _, t=t):
                accv[pl.ds(i * 16, 16)] = (accv[pl.ds(i * 16, 16)]
                                           + tmpv[t, pl.ds(i * 16, 16)])
                return ()
            lax.fori_loop(0, CHW // 16, aloop, ())
        pltpu.sync_copy(accv, out_hbm.at[c, pl.ds(s * CHW, CHW)])

    return k(edge_index)


def _tc_linear(x, W, degp):
    def body(x_ref, w_ref, degp_ref, h_ref, dis_ref):
        deg = degp_ref[0] + degp_ref[1] + 1.0
        dis = lax.rsqrt(deg)
        dis_ref[...] = dis
        discol = jnp.reshape(dis, (NPAD, 1))[:N]
        h = jnp.dot(x_ref[...].astype(jnp.bfloat16),
                    w_ref[...].astype(jnp.bfloat16),
                    preferred_element_type=jnp.float32)
        hs = h * discol
        h_ref[0:N] = hs[:, 0:H]
        h_ref[N:2 * N] = hs[:, H:D]

    return pl.pallas_call(
        body,
        out_shape=(jax.ShapeDtypeStruct((2 * N, H), jnp.float32),
                   jax.ShapeDtypeStruct((NPAD,), jnp.float32)),
    )(x, W, degp)


def _sc_edge_pass(h2, edge_index):
    mesh = plsc.VectorSubcoreMesh(core_axis_name="c", subcore_axis_name="s")

    @functools.partial(
        pl.kernel,
        out_type=jax.ShapeDtypeStruct((NC, N, H), jnp.float32),
        mesh=mesh,
        compiler_params=pltpu.CompilerParams(needs_layout_passes=False),
        scratch_types=[
            pltpu.VMEM((SEC,), jnp.int32),
            pltpu.VMEM((SEC,), jnp.int32),
            pltpu.VMEM((NSLOT, CH), jnp.int32),
            pltpu.VMEM((NSLOT, CH), jnp.int32),
            pltpu.VMEM((NSLOT, CH, H), jnp.float32),
            pltpu.VMEM_SHARED((N, H), jnp.float32),
            pltpu.SemaphoreType.DMA((NSLOT,)),
            pltpu.SemaphoreType.DMA((NSLOT,)),
        ],
    )
    def k(h_hbm, ei_hbm, out_hbm,
          srcv, dstv, idxg, idxs, rows, accS, gsem, ssem):
        c = lax.axis_index("c")
        s = lax.axis_index("s")
        hoff = c * N
        base = s * EPT_C
        zero16 = jnp.zeros((16,), jnp.float32)

        def load_sec(sec):
            o = base + sec * SEC
            pltpu.sync_copy(ei_hbm.at[pl.ds(o, SEC)], srcv)
            pltpu.sync_copy(ei_hbm.at[pl.ds(E + o, SEC)], dstv)

        def build(t, slot):
            loff = (t % CPS) * CH
            for j in range(CH // 16):
                idxg[slot, pl.ds(j * 16, 16)] = (
                    srcv[pl.ds(loff + j * 16, 16)] + hoff)
                idxs[slot, pl.ds(j * 16, 16)] = dstv[pl.ds(loff + j * 16, 16)]

        def issue_gather(slot):
            pltpu.async_copy(h_hbm.at[idxg.at[slot]], rows.at[slot],
                             gsem.at[slot])

        load_sec(0)
        for t0 in range(NSLOT - 1):
            build(t0, t0)
            issue_gather(t0)

        zslot = NSLOT - 1

        def zb(kk, _):
            r = kk // (H // 16)
            jcol = (kk % (H // 16)) * 16
            rows[zslot, r, pl.ds(jcol, 16)] = zero16
            return ()
        lax.fori_loop(0, CH * (H // 16), zb, ())
        for rblk in range(WR_PT):
            off = pl.multiple_of(s * (WR * WR_PT) + rblk * WR, 8)

            @pl.when(off < N)
            def _():
                pltpu.async_copy(rows.at[zslot], accS.at[pl.ds(off, WR)],
                                 ssem.at[zslot])
        for rblk in range(WR_PT):
            off = pl.multiple_of(s * (WR * WR_PT) + rblk * WR, 8)

            @pl.when(off < N)
            def _():
                pltpu.make_async_copy(rows.at[zslot],
                                      accS.at[pl.ds(off, WR)],
                                      ssem.at[zslot]).wait()
        plsc.subcore_barrier()

        def step(t, slot):
            pltpu.make_async_copy(h_hbm.at[idxg.at[slot]], rows.at[slot],
                                  gsem.at[slot]).wait()
            pltpu.async_copy(rows.at[slot], accS.at[idxs.at[slot]],
                             ssem.at[slot], add=True)
            pslot = (slot - 1) % NSLOT

            @pl.when(t >= 1)
            def _():
                pltpu.make_async_copy(rows.at[pslot], accS.at[idxs.at[pslot]],
                                      ssem.at[pslot]).wait()
            nxt = t + NSLOT - 1
            nslot = (slot + NSLOT - 1) % NSLOT

            @pl.when(nxt < NCH)
            def _():
                @pl.when(nxt % CPS == 0)
                def _():
                    load_sec(nxt // CPS)
                build(nxt, nslot)
                issue_gather(nslot)

        @pl.loop(0, NCH - 1, step=NSLOT)
        def _(g):
            for j in range(NSLOT):
                step(g + j, j)

        lastslot = (NCH - 1) % NSLOT
        step(NCH - 1, lastslot)
        pltpu.make_async_copy(rows.at[lastslot], accS.at[idxs.at[lastslot]],
                              ssem.at[lastslot]).wait()

        plsc.subcore_barrier()
        for rblk in range(WR_PT):
            off = pl.multiple_of(s * (WR * WR_PT) + rblk * WR, 8)

            @pl.when(off < N)
            def _():
                pltpu.sync_copy(accS.at[pl.ds(off, WR)],
                                out_hbm.at[c, pl.ds(off, WR)])

    return k(h2, edge_index)


def _tc_epilogue(acc, h2, dis, b, gamma, beta):
    def body(acc_ref, h_ref, dis_ref, b_ref, g_ref, be_ref, o_ref):
        dis_col = jnp.reshape(dis_ref[...], (NPAD, 1))[:N]
        t0 = (acc_ref[0] + h_ref[0:N]) * dis_col
        t1 = (acc_ref[1] + h_ref[N:2 * N]) * dis_col
        t = jnp.concatenate([t0, t1], axis=1) + b_ref[...]
        mu = jnp.mean(t, axis=1, keepdims=True)
        d = t - mu
        var = jnp.mean(d * d, axis=1, keepdims=True)
        o_ref[...] = (d * lax.rsqrt(var + 1e-5)) * g_ref[...] + be_ref[...]

    return pl.pallas_call(
        body,
        out_shape=jax.ShapeDtypeStruct((N, D), jnp.float32),
    )(acc, h2, dis, b, gamma, beta)


def kernel(x, edge_index, W, b, gamma, beta):
    ei = edge_index.astype(jnp.int32).reshape(-1)
    degp = _sc_degree(ei)
    h2, dis = _tc_linear(x, W, degp)
    acc = _sc_edge_pass(h2, ei)
    return _tc_epilogue(acc, h2, dis,
                        b.reshape(1, D), gamma.reshape(1, D),
                        beta.reshape(1, D))


# speedup vs baseline: 27.6242x; 1.0299x over previous
import functools

import jax
import jax.numpy as jnp
from jax import lax
from jax.experimental import pallas as pl
from jax.experimental.pallas import tpu as pltpu
from jax.experimental.pallas import tpu_sc as plsc

N = 10000
E = 160000
D = 256
H = 128
NC = 2
NS = 16
NPAD = 10240
CHW = NPAD // NS
EPT_A = E // (NC * NS)
EPT_C = E // NS
CH = 80
NCH = EPT_C // CH
WR = 80
WR_PT = 8
NSLOT = 4
SEC = 2000
CPS = SEC // CH
NSEC = EPT_C // SEC


def _sc_degree(edge_index):
    mesh = plsc.VectorSubcoreMesh(core_axis_name="c", subcore_axis_name="s")

    @functools.partial(
        pl.kernel,
        out_type=jax.ShapeDtypeStruct((NC, NPAD), jnp.float32),
        mesh=mesh,
        compiler_params=pltpu.CompilerParams(needs_layout_passes=False),
        scratch_types=[
            pltpu.VMEM((EPT_A + 16,), jnp.int32),
            pltpu.VMEM((NPAD,), jnp.float32),
            pltpu.VMEM_SHARED((NS, NPAD), jnp.float32),
            pltpu.VMEM((CHW,), jnp.float32),
            pltpu.VMEM((NS, CHW), jnp.float32),
            pltpu.SemaphoreType.DMA,
        ],
    )
    def k(ei_hbm, out_hbm, idxv, hist, shared, accv, tmpv, csem):
        c = lax.axis_index("c")
        s = lax.axis_index("s")
        tid = c * NS + s
        zero16 = jnp.zeros((16,), jnp.float32)
        zero16i = jnp.zeros((16,), jnp.int32)
        idxv[pl.ds(EPT_A - 8, 16)] = zero16i
        pltpu.sync_copy(ei_hbm.at[pl.ds(E + tid * EPT_A, EPT_A)],
                        idxv.at[pl.ds(0, EPT_A)])

        def zloop(i, _):
            hist[pl.ds(i * 16, 16)] = zero16
            return ()
        lax.fori_loop(0, NPAD // 16, zloop, (), unroll=8)

        ones = jnp.ones((16,), jnp.float32)
        lane = lax.broadcasted_iota(jnp.int32, (16,), 0)

        def hloop(i, _):
            idx = idxv[pl.ds(i * 16, 16)]
            vals = jnp.where(i * 16 + lane < EPT_A, ones, zero16)
            plsc.addupdate_scatter(hist, [idx], vals)
            return ()
        lax.fori_loop(0, (EPT_A + 15) // 16, hloop, (), unroll=8)

        pltpu.sync_copy(hist, shared.at[s])
        plsc.subcore_barrier()

        def z2(i, _):
            accv[pl.ds(i * 16, 16)] = zero16
            return ()
        lax.fori_loop(0, CHW // 16, z2, ())
        for t in range(NS):
            pltpu.async_copy(shared.at[t, pl.ds(s * CHW, CHW)], tmpv.at[t],
                             csem)
        for t in range(NS):
            pltpu.make_async_copy(shared.at[t, pl.ds(s * CHW, CHW)],
                                  tmpv.at[t], csem).wait()

            def aloop(i, _, t=t):
                accv[pl.ds(i * 16, 16)] = (accv[pl.ds(i * 16, 16)]
                                           + tmpv[t, pl.ds(i * 16, 16)])
                return ()
            lax.fori_loop(0, CHW // 16, aloop, (), unroll=8)
        pltpu.sync_copy(accv, out_hbm.at[c, pl.ds(s * CHW, CHW)])

    return k(edge_index)


def _tc_linear(x, W, degp):
    def body(x_ref, w_ref, degp_ref, h_ref, dis_ref):
        deg = degp_ref[0] + degp_ref[1] + 1.0
        dis = lax.rsqrt(deg)
        dis_ref[...] = dis
        discol = jnp.reshape(dis, (NPAD, 1))[:N]
        h = jnp.dot(x_ref[...].astype(jnp.bfloat16),
                    w_ref[...].astype(jnp.bfloat16),
                    preferred_element_type=jnp.float32)
        hs = h * discol
        h_ref[0:N] = hs[:, 0:H]
        h_ref[N:2 * N] = hs[:, H:D]

    return pl.pallas_call(
        body,
        out_shape=(jax.ShapeDtypeStruct((2 * N, H), jnp.float32),
                   jax.ShapeDtypeStruct((NPAD,), jnp.float32)),
    )(x, W, degp)


def _sc_edge_pass(h2, edge_index):
    mesh = plsc.VectorSubcoreMesh(core_axis_name="c", subcore_axis_name="s")

    @functools.partial(
        pl.kernel,
        out_type=jax.ShapeDtypeStruct((NC, N, H), jnp.float32),
        mesh=mesh,
        compiler_params=pltpu.CompilerParams(needs_layout_passes=False),
        scratch_types=[
            pltpu.VMEM((SEC,), jnp.int32),
            pltpu.VMEM((SEC,), jnp.int32),
            pltpu.VMEM((NSLOT, CH), jnp.int32),
            pltpu.VMEM((NSLOT, CH), jnp.int32),
            pltpu.VMEM((NSLOT, CH, H), jnp.float32),
            pltpu.VMEM_SHARED((N, H), jnp.float32),
            pltpu.SemaphoreType.DMA((NSLOT,)),
            pltpu.SemaphoreType.DMA((NSLOT,)),
        ],
    )
    def k(h_hbm, ei_hbm, out_hbm,
          srcv, dstv, idxg, idxs, rows, accS, gsem, ssem):
        c = lax.axis_index("c")
        s = lax.axis_index("s")
        hoff = c * N
        base = s * EPT_C
        zero16 = jnp.zeros((16,), jnp.float32)

        def load_sec(sec):
            o = base + sec * SEC
            pltpu.sync_copy(ei_hbm.at[pl.ds(o, SEC)], srcv)
            pltpu.sync_copy(ei_hbm.at[pl.ds(E + o, SEC)], dstv)

        def build(t, slot):
            loff = (t % CPS) * CH
            for j in range(CH // 16):
                idxg[slot, pl.ds(j * 16, 16)] = (
                    srcv[pl.ds(loff + j * 16, 16)] + hoff)
                idxs[slot, pl.ds(j * 16, 16)] = dstv[pl.ds(loff + j * 16, 16)]

        def issue_gather(slot):
            pltpu.async_copy(h_hbm.at[idxg.at[slot]], rows.at[slot],
                             gsem.at[slot])

        load_sec(0)
        for t0 in range(NSLOT - 1):
            build(t0, t0)
            issue_gather(t0)

        zslot = NSLOT - 1

        def zb(kk, _):
            r = kk // (H // 16)
            jcol = (kk % (H // 16)) * 16
            rows[zslot, r, pl.ds(jcol, 16)] = zero16
            return ()
        lax.fori_loop(0, CH * (H // 16), zb, (), unroll=8)
        for rblk in range(WR_PT):
            off = pl.multiple_of(s * (WR * WR_PT) + rblk * WR, 8)

            @pl.when(off < N)
            def _():
                pltpu.async_copy(rows.at[zslot], accS.at[pl.ds(off, WR)],
                                 ssem.at[zslot])
        for rblk in range(WR_PT):
            off = pl.multiple_of(s * (WR * WR_PT) + rblk * WR, 8)

            @pl.when(off < N)
            def _():
                pltpu.make_async_copy(rows.at[zslot],
                                      accS.at[pl.ds(off, WR)],
                                      ssem.at[zslot]).wait()
        plsc.subcore_barrier()

        def step(t, slot):
            pltpu.make_async_copy(h_hbm.at[idxg.at[slot]], rows.at[slot],
                                  gsem.at[slot]).wait()
            pltpu.async_copy(rows.at[slot], accS.at[idxs.at[slot]],
                             ssem.at[slot], add=True)
            pslot = (slot - 1) % NSLOT

            @pl.when(t >= 1)
            def _():
                pltpu.make_async_copy(rows.at[pslot], accS.at[idxs.at[pslot]],
                                      ssem.at[pslot]).wait()
            nxt = t + NSLOT - 1
            nslot = (slot + NSLOT - 1) % NSLOT

            @pl.when(nxt < NCH)
            def _():
                @pl.when(nxt % CPS == 0)
                def _():
                    load_sec(nxt // CPS)
                build(nxt, nslot)
                issue_gather(nslot)

        @pl.loop(0, NCH - 1, step=NSLOT)
        def _(g):
            for j in range(NSLOT):
                step(g + j, j)

        lastslot = (NCH - 1) % NSLOT
        step(NCH - 1, lastslot)
        pltpu.make_async_copy(rows.at[lastslot], accS.at[idxs.at[lastslot]],
                              ssem.at[lastslot]).wait()

        plsc.subcore_barrier()
        for rblk in range(WR_PT):
            off = pl.multiple_of(s * (WR * WR_PT) + rblk * WR, 8)

            @pl.when(off < N)
            def _():
                pltpu.sync_copy(accS.at[pl.ds(off, WR)],
                                out_hbm.at[c, pl.ds(off, WR)])

    return k(h2, edge_index)


def _tc_epilogue(acc, h2, dis, b, gamma, beta):
    def body(acc_ref, h_ref, dis_ref, b_ref, g_ref, be_ref, o_ref):
        dis_col = jnp.reshape(dis_ref[...], (NPAD, 1))[:N]
        t0 = (acc_ref[0] + h_ref[0:N]) * dis_col
        t1 = (acc_ref[1] + h_ref[N:2 * N]) * dis_col
        t = jnp.concatenate([t0, t1], axis=1) + b_ref[...]
        mu = jnp.mean(t, axis=1, keepdims=True)
        d = t - mu
        var = jnp.mean(d * d, axis=1, keepdims=True)
        o_ref[...] = (d * lax.rsqrt(var + 1e-5)) * g_ref[...] + be_ref[...]

    return pl.pallas_call(
        body,
        out_shape=jax.ShapeDtypeStruct((N, D), jnp.float32),
    )(acc, h2, dis, b, gamma, beta)


def kernel(x, edge_index, W, b, gamma, beta):
    ei = edge_index.astype(jnp.int32).reshape(-1)
    degp = _sc_degree(ei)
    h2, dis = _tc_linear(x, W, degp)
    acc = _sc_edge_pass(h2, ei)
    return _tc_epilogue(acc, h2, dis,
                        b.reshape(1, D), gamma.reshape(1, D),
                        beta.reshape(1, D))


# speedup vs baseline: 27.8425x; 1.0079x over previous
import functools

import jax
import jax.numpy as jnp
from jax import lax
from jax.experimental import pallas as pl
from jax.experimental.pallas import tpu as pltpu
from jax.experimental.pallas import tpu_sc as plsc

N = 10000
E = 160000
D = 256
H = 128
NC = 2
NS = 16
NPAD = 10240
CHW = NPAD // NS
EPT_A = E // (NC * NS)
EPT_C = E // NS
CH = 80
NCH = EPT_C // CH
WR = 80
WR_PT = 8
NSLOT = 4
SEC = 2000
CPS = SEC // CH
NSEC = EPT_C // SEC
WIN_A = 5248
SECW = 2176


def _sc_degree(edge_index):
    mesh = plsc.VectorSubcoreMesh(core_axis_name="c", subcore_axis_name="s")

    @functools.partial(
        pl.kernel,
        out_type=jax.ShapeDtypeStruct((NC, NPAD), jnp.float32),
        mesh=mesh,
        compiler_params=pltpu.CompilerParams(needs_layout_passes=False),
        scratch_types=[
            pltpu.VMEM((2, WIN_A), jnp.int32),
            pltpu.VMEM((NPAD,), jnp.float32),
            pltpu.VMEM_SHARED((NS, NPAD), jnp.float32),
            pltpu.VMEM((CHW,), jnp.float32),
            pltpu.VMEM((NS, CHW), jnp.float32),
            pltpu.SemaphoreType.DMA,
        ],
    )
    def k(ei_hbm, out_hbm, idxv, hist, shared, accv, tmpv, csem):
        c = lax.axis_index("c")
        s = lax.axis_index("s")
        tid = c * NS + s
        zero16 = jnp.zeros((16,), jnp.float32)
        o = tid * EPT_A
        o_cl = jnp.minimum((o // 128) * 128, E - WIN_A)
        lo = o - o_cl
        pltpu.sync_copy(ei_hbm.at[pl.ds(0, 2), pl.ds(o_cl, WIN_A)], idxv)

        def zloop(i, _):
            hist[pl.ds(i * 16, 16)] = zero16
            return ()
        lax.fori_loop(0, NPAD // 16, zloop, (), unroll=8)

        ones = jnp.ones((16,), jnp.float32)
        lane = lax.broadcasted_iota(jnp.int32, (16,), 0)

        def hloop(i, _):
            idx = idxv[1, pl.ds(i * 16, 16)]
            pos = i * 16 + lane
            vals = jnp.where((pos >= lo) & (pos < lo + EPT_A), ones, zero16)
            plsc.addupdate_scatter(hist, [idx], vals)
            return ()
        lax.fori_loop(0, WIN_A // 16, hloop, (), unroll=8)

        pltpu.sync_copy(hist, shared.at[s])
        plsc.subcore_barrier()

        def z2(i, _):
            accv[pl.ds(i * 16, 16)] = zero16
            return ()
        lax.fori_loop(0, CHW // 16, z2, ())
        for t in range(NS):
            pltpu.async_copy(shared.at[t, pl.ds(s * CHW, CHW)], tmpv.at[t],
                             csem)
        for t in range(NS):
            pltpu.make_async_copy(shared.at[t, pl.ds(s * CHW, CHW)],
                                  tmpv.at[t], csem).wait()

            def aloop(i, _, t=t):
                accv[pl.ds(i * 16, 16)] = (accv[pl.ds(i * 16, 16)]
                                           + tmpv[t, pl.ds(i * 16, 16)])
                return ()
            lax.fori_loop(0, CHW // 16, aloop, (), unroll=8)
        pltpu.sync_copy(accv, out_hbm.at[c, pl.ds(s * CHW, CHW)])

    return k(edge_index)


def _tc_linear(x, W, degp):
    def body(x_ref, w_ref, degp_ref, h_ref, dis_ref):
        deg = degp_ref[0] + degp_ref[1] + 1.0
        dis = lax.rsqrt(deg)
        dis_ref[...] = dis
        discol = jnp.reshape(dis, (NPAD, 1))[:N]
        h = jnp.dot(x_ref[...].astype(jnp.bfloat16),
                    w_ref[...].astype(jnp.bfloat16),
                    preferred_element_type=jnp.float32)
        hs = h * discol
        h_ref[0:N] = hs[:, 0:H]
        h_ref[N:2 * N] = hs[:, H:D]

    return pl.pallas_call(
        body,
        out_shape=(jax.ShapeDtypeStruct((2 * N, H), jnp.float32),
                   jax.ShapeDtypeStruct((NPAD,), jnp.float32)),
    )(x, W, degp)


def _sc_edge_pass(h2, edge_index):
    mesh = plsc.VectorSubcoreMesh(core_axis_name="c", subcore_axis_name="s")

    @functools.partial(
        pl.kernel,
        out_type=jax.ShapeDtypeStruct((NC, N, H), jnp.float32),
        mesh=mesh,
        compiler_params=pltpu.CompilerParams(needs_layout_passes=False),
        scratch_types=[
            pltpu.VMEM((2, SECW), jnp.int32),
            pltpu.VMEM((NSLOT, CH), jnp.int32),
            pltpu.VMEM((NSLOT, CH), jnp.int32),
            pltpu.VMEM((NSLOT, CH, H), jnp.float32),
            pltpu.VMEM_SHARED((N, H), jnp.float32),
            pltpu.SemaphoreType.DMA((NSLOT,)),
            pltpu.SemaphoreType.DMA((NSLOT,)),
        ],
    )
    def k(h_hbm, ei_hbm, out_hbm,
          secv, idxg, idxs, rows, accS, gsem, ssem):
        c = lax.axis_index("c")
        s = lax.axis_index("s")
        hoff = c * N
        base = s * EPT_C
        zero16 = jnp.zeros((16,), jnp.float32)

        def sec_start(sec):
            o = base + sec * SEC
            return o, jnp.minimum((o // 128) * 128, E - SECW)

        def load_sec(sec):
            _, o_cl = sec_start(sec)
            pltpu.sync_copy(ei_hbm.at[pl.ds(0, 2), pl.ds(o_cl, SECW)], secv)

        def build(t, slot):
            o, o_cl = sec_start(t // CPS)
            loff = (o - o_cl) + (t % CPS) * CH
            for j in range(CH // 16):
                idxg[slot, pl.ds(j * 16, 16)] = (
                    secv[0, pl.ds(loff + j * 16, 16)] + hoff)
                idxs[slot, pl.ds(j * 16, 16)] = secv[1, pl.ds(loff + j * 16, 16)]

        def issue_gather(slot):
            pltpu.async_copy(h_hbm.at[idxg.at[slot]], rows.at[slot],
                             gsem.at[slot])

        load_sec(0)
        for t0 in range(NSLOT - 1):
            build(t0, t0)
            issue_gather(t0)

        zslot = NSLOT - 1

        def zb(kk, _):
            r = kk // (H // 16)
            jcol = (kk % (H // 16)) * 16
            rows[zslot, r, pl.ds(jcol, 16)] = zero16
            return ()
        lax.fori_loop(0, CH * (H // 16), zb, (), unroll=8)
        for rblk in range(WR_PT):
            off = pl.multiple_of(s * (WR * WR_PT) + rblk * WR, 8)

            @pl.when(off < N)
            def _():
                pltpu.async_copy(rows.at[zslot], accS.at[pl.ds(off, WR)],
                                 ssem.at[zslot])
        for rblk in range(WR_PT):
            off = pl.multiple_of(s * (WR * WR_PT) + rblk * WR, 8)

            @pl.when(off < N)
            def _():
                pltpu.make_async_copy(rows.at[zslot],
                                      accS.at[pl.ds(off, WR)],
                                      ssem.at[zslot]).wait()
        plsc.subcore_barrier()

        def step(t, slot):
            pltpu.make_async_copy(h_hbm.at[idxg.at[slot]], rows.at[slot],
                                  gsem.at[slot]).wait()
            pltpu.async_copy(rows.at[slot], accS.at[idxs.at[slot]],
                             ssem.at[slot], add=True)
            pslot = (slot - 1) % NSLOT

            @pl.when(t >= 1)
            def _():
                pltpu.make_async_copy(rows.at[pslot], accS.at[idxs.at[pslot]],
                                      ssem.at[pslot]).wait()
            nxt = t + NSLOT - 1
            nslot = (slot + NSLOT - 1) % NSLOT

            @pl.when(nxt < NCH)
            def _():
                @pl.when(nxt % CPS == 0)
                def _():
                    load_sec(nxt // CPS)
                build(nxt, nslot)
                issue_gather(nslot)

        @pl.loop(0, NCH - 1, step=NSLOT)
        def _(g):
            for j in range(NSLOT):
                step(g + j, j)

        lastslot = (NCH - 1) % NSLOT
        step(NCH - 1, lastslot)
        pltpu.make_async_copy(rows.at[lastslot], accS.at[idxs.at[lastslot]],
                              ssem.at[lastslot]).wait()

        plsc.subcore_barrier()
        for rblk in range(WR_PT):
            off = pl.multiple_of(s * (WR * WR_PT) + rblk * WR, 8)

            @pl.when(off < N)
            def _():
                pltpu.sync_copy(accS.at[pl.ds(off, WR)],
                                out_hbm.at[c, pl.ds(off, WR)])

    return k(h2, edge_index)


def _tc_epilogue(acc, h2, dis, b, gamma, beta):
    def body(acc_ref, h_ref, dis_ref, b_ref, g_ref, be_ref, o_ref):
        dis_col = jnp.reshape(dis_ref[...], (NPAD, 1))[:N]
        t0 = (acc_ref[0] + h_ref[0:N]) * dis_col
        t1 = (acc_ref[1] + h_ref[N:2 * N]) * dis_col
        t = jnp.concatenate([t0, t1], axis=1) + b_ref[...]
        mu = jnp.mean(t, axis=1, keepdims=True)
        d = t - mu
        var = jnp.mean(d * d, axis=1, keepdims=True)
        o_ref[...] = (d * lax.rsqrt(var + 1e-5)) * g_ref[...] + be_ref[...]

    return pl.pallas_call(
        body,
        out_shape=jax.ShapeDtypeStruct((N, D), jnp.float32),
    )(acc, h2, dis, b, gamma, beta)


def kernel(x, edge_index, W, b, gamma, beta):
    ei = edge_index.astype(jnp.int32)
    degp = _sc_degree(ei)
    h2, dis = _tc_linear(x, W, degp)
    acc = _sc_edge_pass(h2, ei)
    return _tc_epilogue(acc, h2, dis,
                        b.reshape(1, D), gamma.reshape(1, D),
                        beta.reshape(1, D))
